# Initial kernel scaffold; baseline (speedup 1.0000x reference)
import functools

import jax
import jax.numpy as jnp
import numpy as np
from jax import lax
from jax.experimental import pallas as pl
from jax.experimental.pallas import tpu as pltpu
from jax.experimental.pallas import tpu_sc as plsc

_LIST_OF_N = [100003, 100019, 100043, 100049]
_D = 128
_DM = 2048
_K = 4
_B, _L, _H = 4, 4096, 4
_HD = _H * _D
_ROWS = _B * _L * _H
_NW = 32
_RPW = _ROWS // _NW
_CH = 256
_NCH = _RPW // _CH

_TL = 512
_NL = _L // _TL


def _gather_sc(ids3, table):
  mesh = plsc.VectorSubcoreMesh(core_axis_name="c", subcore_axis_name="s")

  @functools.partial(
      pl.kernel,
      mesh=mesh,
      out_type=jax.ShapeDtypeStruct((_ROWS, _D), jnp.float32),
      scratch_types=[
          pltpu.VMEM((_NCH, _CH), jnp.int32),
          pltpu.VMEM((_CH, _D), jnp.float32),
          pltpu.VMEM((_CH, _D), jnp.float32),
          pltpu.SemaphoreType.DMA,
          pltpu.SemaphoreType.DMA,
      ],
  )
  def k(ids_hbm, table_hbm, out_hbm, idx_v, buf0, buf1, sem0, sem1):
    wid = lax.axis_index("s") * 2 + lax.axis_index("c")
    base = wid * _RPW
    pltpu.sync_copy(ids_hbm.at[wid], idx_v)
    bufs = (buf0, buf1)
    sems = (sem0, sem1)
    cps = [None, None]
    cps[0] = pltpu.async_copy(table_hbm.at[idx_v.at[0]], buf0, sem0)
    for c in range(_NCH):
      cur = c % 2
      if c + 1 < _NCH:
        nxt = (c + 1) % 2
        cps[nxt] = pltpu.async_copy(
            table_hbm.at[idx_v.at[c + 1]], bufs[nxt], sems[nxt])
      cps[cur].wait()
      pltpu.sync_copy(bufs[cur], out_hbm.at[pl.ds(base + c * _CH, _CH)])

  return k(ids3, table)


def _tc_body(x_ref, cw_ref, w_ref, out_ref, carry_ref):
  il = pl.program_id(1)

  @pl.when(il == 0)
  def _():
    carry_ref[...] = jnp.zeros_like(carry_ref)

  x = x_ref[0]
  prev = carry_ref[...]
  cw = cw_ref[...]
  conv = x * cw[_K - 1][None, :]
  for s in range(1, _K):
    shifted = jnp.concatenate([prev[8 - s:], x[:_TL - s]], axis=0)
    conv = conv + shifted * cw[_K - 1 - s][None, :]
  carry_ref[...] = x[_TL - 8:]
  y = (conv * jax.nn.sigmoid(conv) * x).astype(jnp.bfloat16)
  out_ref[0] = jnp.dot(y, w_ref[...], preferred_element_type=jnp.float32)


def _tc_call(x, conv_w, w_bf16):
  return pl.pallas_call(
      _tc_body,
      grid=(_B, _NL),
      in_specs=[
          pl.BlockSpec((1, _TL, _HD), lambda b, i: (b, i, 0)),
          pl.BlockSpec((_K, _HD), lambda b, i: (0, 0)),
          pl.BlockSpec((_HD, _DM), lambda b, i: (0, 0)),
      ],
      out_specs=pl.BlockSpec((1, _TL, _DM), lambda b, i: (b, i, 0)),
      out_shape=jax.ShapeDtypeStruct((_B, _L, _DM), jnp.float32),
      scratch_shapes=[pltpu.VMEM((8, _HD), jnp.float32)],
      compiler_params=pltpu.CompilerParams(
          dimension_semantics=("arbitrary", "arbitrary")),
  )(x, conv_w, w_bf16)


def kernel(input_ids, emb_table, conv_w, w_out):
  offsets = jnp.array(np.cumsum([0] + _LIST_OF_N[:-1]), dtype=input_ids.dtype)
  shifted = (input_ids + offsets[None, None, :]).reshape(_NW, _NCH, _CH)
  rows = _gather_sc(shifted, emb_table)
  x = rows.reshape(_B, _L, _HD)
  return _tc_call(x, conv_w, w_out.astype(jnp.bfloat16))


# speedup vs baseline: 1.3259x; 1.3259x over previous
import functools

import jax
import jax.numpy as jnp
import numpy as np
from jax import lax
from jax.experimental import pallas as pl
from jax.experimental.pallas import tpu as pltpu
from jax.experimental.pallas import tpu_sc as plsc

_LIST_OF_N = [100003, 100019, 100043, 100049]
_D = 128
_DM = 2048
_K = 4
_B, _L, _H = 4, 4096, 4
_HD = _H * _D
_ROWS = _B * _L * _H
_NW = 32
_RPW = _ROWS // _NW
_CH = 128
_NCH = _RPW // _CH

_TL = 512
_NL = _L // _TL


def _gather_sc(ids3, table):
  mesh = plsc.VectorSubcoreMesh(core_axis_name="c", subcore_axis_name="s")

  @functools.partial(
      pl.kernel,
      mesh=mesh,
      out_type=jax.ShapeDtypeStruct((_ROWS, _D), jnp.float32),
      scratch_types=[
          pltpu.VMEM((_NCH, _CH), jnp.int32),
          pltpu.VMEM((_CH, _D), jnp.float32),
          pltpu.VMEM((_CH, _D), jnp.float32),
          pltpu.SemaphoreType.DMA,
          pltpu.SemaphoreType.DMA,
      ],
  )
  def k(ids_hbm, table_hbm, out_hbm, idx_v, buf0, buf1, sem0, sem1):
    wid = lax.axis_index("s") * 2 + lax.axis_index("c")
    base = wid * _RPW
    pltpu.sync_copy(ids_hbm.at[wid], idx_v)
    bufs = (buf0, buf1)
    sems = (sem0, sem1)
    cps = [None, None]
    cps[0] = pltpu.async_copy(table_hbm.at[idx_v.at[0]], buf0, sem0)
    for c in range(_NCH):
      cur = c % 2
      if c + 1 < _NCH:
        nxt = (c + 1) % 2
        cps[nxt] = pltpu.async_copy(
            table_hbm.at[idx_v.at[c + 1]], bufs[nxt], sems[nxt])
      cps[cur].wait()
      pltpu.sync_copy(bufs[cur], out_hbm.at[pl.ds(base + c * _CH, _CH)])

  return k(ids3, table)


def _tc_body(x_ref, cw_ref, w_ref, out_ref, carry_ref):
  il = pl.program_id(1)

  @pl.when(il == 0)
  def _():
    carry_ref[...] = jnp.zeros_like(carry_ref)

  x = x_ref[0]
  prev = carry_ref[...]
  cw = cw_ref[...]
  conv = x * cw[_K - 1][None, :]
  for s in range(1, _K):
    shifted = jnp.concatenate([prev[8 - s:], x[:_TL - s]], axis=0)
    conv = conv + shifted * cw[_K - 1 - s][None, :]
  carry_ref[...] = x[_TL - 8:]
  y = (conv * jax.nn.sigmoid(conv) * x).astype(jnp.bfloat16)
  out_ref[0] = jnp.dot(y, w_ref[...], preferred_element_type=jnp.float32)


def _tc_call(x, conv_w, w_bf16):
  return pl.pallas_call(
      _tc_body,
      grid=(_B, _NL),
      in_specs=[
          pl.BlockSpec((1, _TL, _HD), lambda b, i: (b, i, 0)),
          pl.BlockSpec((_K, _HD), lambda b, i: (0, 0)),
          pl.BlockSpec((_HD, _DM), lambda b, i: (0, 0)),
      ],
      out_specs=pl.BlockSpec((1, _TL, _DM), lambda b, i: (b, i, 0)),
      out_shape=jax.ShapeDtypeStruct((_B, _L, _DM), jnp.float32),
      scratch_shapes=[pltpu.VMEM((8, _HD), jnp.float32)],
      compiler_params=pltpu.CompilerParams(
          dimension_semantics=("arbitrary", "arbitrary")),
  )(x, conv_w, w_bf16)


def kernel(input_ids, emb_table, conv_w, w_out):
  offsets = jnp.array(np.cumsum([0] + _LIST_OF_N[:-1]), dtype=input_ids.dtype)
  shifted = (input_ids + offsets[None, None, :]).reshape(_NW, _NCH, _CH)
  rows = _gather_sc(shifted, emb_table)
  x = rows.reshape(_B, _L, _HD)
  return _tc_call(x, conv_w, w_out.astype(jnp.bfloat16))


# speedup vs baseline: 2.0281x; 1.5296x over previous
import functools

import jax
import jax.numpy as jnp
import numpy as np
from jax import lax
from jax.experimental import pallas as pl
from jax.experimental.pallas import tpu as pltpu
from jax.experimental.pallas import tpu_sc as plsc

_LIST_OF_N = [100003, 100019, 100043, 100049]
_D = 128
_DM = 2048
_K = 4
_B, _L, _H = 4, 4096, 4
_HD = _H * _D
_BL = _B * _L
_ROWS = _B * _L * _H
_NW = 32
_RPW = _ROWS // _NW
_WPH = _NW // _H
_CH = 128
_NCH = _RPW // _CH

_TL = 1024
_NBL = _BL // _TL
_BPB = _L // _TL


def _gather_sc(ids3, table):
  mesh = plsc.VectorSubcoreMesh(core_axis_name="c", subcore_axis_name="s")

  @functools.partial(
      pl.kernel,
      mesh=mesh,
      out_type=jax.ShapeDtypeStruct((_H, _BL, _D), jnp.float32),
      scratch_types=[
          pltpu.VMEM((_NCH, _CH), jnp.int32),
          pltpu.VMEM((_CH, _D), jnp.float32),
          pltpu.VMEM((_CH, _D), jnp.float32),
          pltpu.SemaphoreType.DMA,
          pltpu.SemaphoreType.DMA,
      ],
  )
  def k(ids_hbm, table_hbm, out_hbm, idx_v, buf0, buf1, sem0, sem1):
    wid = lax.axis_index("s") * 2 + lax.axis_index("c")
    h = wid // _WPH
    base = (wid % _WPH) * _RPW
    pltpu.sync_copy(ids_hbm.at[wid], idx_v)
    bufs = (buf0, buf1)
    sems = (sem0, sem1)
    cps = [None, None]
    cps[0] = pltpu.async_copy(table_hbm.at[idx_v.at[0]], buf0, sem0)
    for c in range(_NCH):
      cur = c % 2
      if c + 1 < _NCH:
        nxt = (c + 1) % 2
        cps[nxt] = pltpu.async_copy(
            table_hbm.at[idx_v.at[c + 1]], bufs[nxt], sems[nxt])
      cps[cur].wait()
      pltpu.sync_copy(bufs[cur], out_hbm.at[h, pl.ds(base + c * _CH, _CH)])

  return k(ids3, table)


def _tc_body(x_ref, cw_ref, w_ref, out_ref, carry_ref):
  j = pl.program_id(0)

  @pl.when(j % _BPB == 0)
  def _():
    carry_ref[...] = jnp.zeros_like(carry_ref)

  x4 = x_ref[...]
  prev = carry_ref[...]
  cw = cw_ref[...]
  conv = x4 * cw[:, _K - 1, :][:, None, :]
  for s in range(1, _K):
    shifted = jnp.concatenate(
        [prev[:, 8 - s:, :], x4[:, :_TL - s, :]], axis=1)
    conv = conv + shifted * cw[:, _K - 1 - s, :][:, None, :]
  carry_ref[...] = x4[:, _TL - 8:, :]
  y4 = (conv * jax.nn.sigmoid(conv) * x4).astype(jnp.bfloat16)
  y = jnp.concatenate([y4[0], y4[1], y4[2], y4[3]], axis=1)
  out_ref[...] = jnp.dot(y, w_ref[...], preferred_element_type=jnp.float32)


def _tc_call(x4, cw4, w_bf16):
  return pl.pallas_call(
      _tc_body,
      grid=(_NBL,),
      in_specs=[
          pl.BlockSpec((_H, _TL, _D), lambda j: (0, j, 0)),
          pl.BlockSpec((_H, _K, _D), lambda j: (0, 0, 0)),
          pl.BlockSpec((_HD, _DM), lambda j: (0, 0)),
      ],
      out_specs=pl.BlockSpec((_TL, _DM), lambda j: (j, 0)),
      out_shape=jax.ShapeDtypeStruct((_BL, _DM), jnp.float32),
      scratch_shapes=[pltpu.VMEM((_H, 8, _D), jnp.float32)],
      compiler_params=pltpu.CompilerParams(
          dimension_semantics=("arbitrary",)),
  )(x4, cw4, w_bf16)


def kernel(input_ids, emb_table, conv_w, w_out):
  offsets = jnp.array(np.cumsum([0] + _LIST_OF_N[:-1]), dtype=input_ids.dtype)
  shifted = (input_ids + offsets[None, None, :]).transpose(2, 0, 1)
  ids3 = shifted.reshape(_NW, _NCH, _CH)
  x4 = _gather_sc(ids3, emb_table)
  cw4 = conv_w.reshape(_K, _H, _D).transpose(1, 0, 2)
  out = _tc_call(x4, cw4, w_out.astype(jnp.bfloat16))
  return out.reshape(_B, _L, _DM)


# speedup vs baseline: 2.0333x; 1.0026x over previous
import functools

import jax
import jax.numpy as jnp
import numpy as np
from jax import lax
from jax.experimental import pallas as pl
from jax.experimental.pallas import tpu as pltpu
from jax.experimental.pallas import tpu_sc as plsc

_LIST_OF_N = [100003, 100019, 100043, 100049]
_D = 128
_DM = 2048
_K = 4
_B, _L, _H = 4, 4096, 4
_HD = _H * _D
_BL = _B * _L
_ROWS = _B * _L * _H
_NW = 32
_RPW = _ROWS // _NW
_WPH = _NW // _H
_CH = 128
_NCH = _RPW // _CH

_TL = 2048
_NBL = _BL // _TL
_BPB = _L // _TL


def _gather_sc(ids3, table):
  mesh = plsc.VectorSubcoreMesh(core_axis_name="c", subcore_axis_name="s")

  @functools.partial(
      pl.kernel,
      mesh=mesh,
      out_type=jax.ShapeDtypeStruct((_H, _BL, _D), jnp.float32),
      scratch_types=[
          pltpu.VMEM((_NCH, _CH), jnp.int32),
          pltpu.VMEM((_CH, _D), jnp.float32),
          pltpu.VMEM((_CH, _D), jnp.float32),
          pltpu.SemaphoreType.DMA,
          pltpu.SemaphoreType.DMA,
      ],
  )
  def k(ids_hbm, table_hbm, out_hbm, idx_v, buf0, buf1, sem0, sem1):
    wid = lax.axis_index("s") * 2 + lax.axis_index("c")
    h = wid // _WPH
    base = (wid % _WPH) * _RPW
    pltpu.sync_copy(ids_hbm.at[wid], idx_v)
    bufs = (buf0, buf1)
    sems = (sem0, sem1)
    cps = [None, None]
    cps[0] = pltpu.async_copy(table_hbm.at[idx_v.at[0]], buf0, sem0)
    for c in range(_NCH):
      cur = c % 2
      if c + 1 < _NCH:
        nxt = (c + 1) % 2
        cps[nxt] = pltpu.async_copy(
            table_hbm.at[idx_v.at[c + 1]], bufs[nxt], sems[nxt])
      cps[cur].wait()
      pltpu.sync_copy(bufs[cur], out_hbm.at[h, pl.ds(base + c * _CH, _CH)])

  return k(ids3, table)


def _tc_body(x_ref, cw_ref, w_ref, out_ref, carry_ref):
  j = pl.program_id(0)

  @pl.when(j % _BPB == 0)
  def _():
    carry_ref[...] = jnp.zeros_like(carry_ref)

  x4 = x_ref[...]
  prev = carry_ref[...]
  cw = cw_ref[...]
  conv = x4 * cw[:, _K - 1, :][:, None, :]
  for s in range(1, _K):
    shifted = jnp.concatenate(
        [prev[:, 8 - s:, :], x4[:, :_TL - s, :]], axis=1)
    conv = conv + shifted * cw[:, _K - 1 - s, :][:, None, :]
  carry_ref[...] = x4[:, _TL - 8:, :]
  y4 = (conv * jax.nn.sigmoid(conv) * x4).astype(jnp.bfloat16)
  y = jnp.concatenate([y4[0], y4[1], y4[2], y4[3]], axis=1)
  out_ref[...] = jnp.dot(y, w_ref[...], preferred_element_type=jnp.float32)


def _tc_call(x4, cw4, w_bf16):
  return pl.pallas_call(
      _tc_body,
      grid=(_NBL,),
      in_specs=[
          pl.BlockSpec((_H, _TL, _D), lambda j: (0, j, 0)),
          pl.BlockSpec((_H, _K, _D), lambda j: (0, 0, 0)),
          pl.BlockSpec((_HD, _DM), lambda j: (0, 0)),
      ],
      out_specs=pl.BlockSpec((_TL, _DM), lambda j: (j, 0)),
      out_shape=jax.ShapeDtypeStruct((_BL, _DM), jnp.float32),
      scratch_shapes=[pltpu.VMEM((_H, 8, _D), jnp.float32)],
      compiler_params=pltpu.CompilerParams(
          dimension_semantics=("arbitrary",)),
  )(x4, cw4, w_bf16)


def kernel(input_ids, emb_table, conv_w, w_out):
  offsets = jnp.array(np.cumsum([0] + _LIST_OF_N[:-1]), dtype=input_ids.dtype)
  shifted = (input_ids + offsets[None, None, :]).transpose(2, 0, 1)
  ids3 = shifted.reshape(_NW, _NCH, _CH)
  x4 = _gather_sc(ids3, emb_table)
  cw4 = conv_w.reshape(_K, _H, _D).transpose(1, 0, 2)
  out = _tc_call(x4, cw4, w_out.astype(jnp.bfloat16))
  return out.reshape(_B, _L, _DM)


# speedup vs baseline: 2.0442x; 1.0054x over previous
import functools

import jax
import jax.numpy as jnp
import numpy as np
from jax import lax
from jax.experimental import pallas as pl
from jax.experimental.pallas import tpu as pltpu
from jax.experimental.pallas import tpu_sc as plsc

_LIST_OF_N = [100003, 100019, 100043, 100049]
_D = 128
_DM = 2048
_K = 4
_B, _L, _H = 4, 4096, 4
_HD = _H * _D
_BL = _B * _L
_ROWS = _B * _L * _H
_NW = 32
_RPW = _ROWS // _NW
_WPH = _NW // _H
_CH = 128
_NCH = _RPW // _CH

_TL = 2048
_NBL = _BL // _TL
_BPB = _L // _TL


def _gather_sc(ids3, table):
  mesh = plsc.VectorSubcoreMesh(core_axis_name="c", subcore_axis_name="s")

  @functools.partial(
      pl.kernel,
      mesh=mesh,
      out_type=jax.ShapeDtypeStruct((_H, _BL, _D), jnp.float32),
      scratch_types=[
          pltpu.VMEM((_NCH, _CH), jnp.int32),
          pltpu.VMEM((_CH, _D), jnp.float32),
          pltpu.VMEM((_CH, _D), jnp.float32),
          pltpu.VMEM((_CH, _D), jnp.float32),
          pltpu.VMEM((_CH, _D), jnp.float32),
          pltpu.SemaphoreType.DMA,
          pltpu.SemaphoreType.DMA,
          pltpu.SemaphoreType.DMA,
          pltpu.SemaphoreType.DMA,
          pltpu.SemaphoreType.DMA,
          pltpu.SemaphoreType.DMA,
          pltpu.SemaphoreType.DMA,
          pltpu.SemaphoreType.DMA,
      ],
  )
  def k(ids_hbm, table_hbm, out_hbm, idx_v, b0, b1, b2, b3,
        g0, g1, g2, g3, w0, w1, w2, w3):
    wid = lax.axis_index("s") * 2 + lax.axis_index("c")
    h = wid // _WPH
    base = (wid % _WPH) * _RPW
    pltpu.sync_copy(ids_hbm.at[wid], idx_v)
    bufs = (b0, b1, b2, b3)
    gsem = (g0, g1, g2, g3)
    wsem = (w0, w1, w2, w3)
    g = [None] * 4
    w = [None] * 4
    for c in range(4):
      g[c] = pltpu.async_copy(table_hbm.at[idx_v.at[c]], bufs[c], gsem[c])
    for c in range(_NCH):
      s = c % 4
      if 2 <= c and c + 2 < _NCH:
        ps = (c - 2) % 4
        w[ps].wait()
        g[ps] = pltpu.async_copy(
            table_hbm.at[idx_v.at[c + 2]], bufs[ps], gsem[ps])
      g[s].wait()
      w[s] = pltpu.async_copy(
          bufs[s], out_hbm.at[h, pl.ds(base + c * _CH, _CH)], wsem[s])
    for s in range(4):
      w[s].wait()

  return k(ids3, table)


def _tc_body(x_ref, cw_ref, w_ref, out_ref, carry_ref):
  j = pl.program_id(0)

  @pl.when(j % _BPB == 0)
  def _():
    carry_ref[...] = jnp.zeros_like(carry_ref)

  x4 = x_ref[...]
  prev = carry_ref[...]
  cw = cw_ref[...]
  conv = x4 * cw[:, _K - 1, :][:, None, :]
  for s in range(1, _K):
    shifted = jnp.concatenate(
        [prev[:, 8 - s:, :], x4[:, :_TL - s, :]], axis=1)
    conv = conv + shifted * cw[:, _K - 1 - s, :][:, None, :]
  carry_ref[...] = x4[:, _TL - 8:, :]
  y4 = (conv * jax.nn.sigmoid(conv) * x4).astype(jnp.bfloat16)
  y = jnp.concatenate([y4[0], y4[1], y4[2], y4[3]], axis=1)
  out_ref[...] = jnp.dot(y, w_ref[...], preferred_element_type=jnp.float32)


def _tc_call(x4, cw4, w_bf16):
  return pl.pallas_call(
      _tc_body,
      grid=(_NBL,),
      in_specs=[
          pl.BlockSpec((_H, _TL, _D), lambda j: (0, j, 0)),
          pl.BlockSpec((_H, _K, _D), lambda j: (0, 0, 0)),
          pl.BlockSpec((_HD, _DM), lambda j: (0, 0)),
      ],
      out_specs=pl.BlockSpec((_TL, _DM), lambda j: (j, 0)),
      out_shape=jax.ShapeDtypeStruct((_BL, _DM), jnp.float32),
      scratch_shapes=[pltpu.VMEM((_H, 8, _D), jnp.float32)],
      compiler_params=pltpu.CompilerParams(
          dimension_semantics=("arbitrary",)),
  )(x4, cw4, w_bf16)


def kernel(input_ids, emb_table, conv_w, w_out):
  offsets = jnp.array(np.cumsum([0] + _LIST_OF_N[:-1]), dtype=input_ids.dtype)
  shifted = (input_ids + offsets[None, None, :]).transpose(2, 0, 1)
  ids3 = shifted.reshape(_NW, _NCH, _CH)
  x4 = _gather_sc(ids3, emb_table)
  cw4 = conv_w.reshape(_K, _H, _D).transpose(1, 0, 2)
  out = _tc_call(x4, cw4, w_out.astype(jnp.bfloat16))
  return out.reshape(_B, _L, _DM)
